# Initial kernel scaffold; baseline (speedup 1.0000x reference)
import jax
import jax.numpy as jnp
from jax.experimental import pallas as pl


def kernel(x, edge_index, W_self1, W_neigh1, b1, W_self2, W_neigh2, b2):
    raise NotImplementedError("write your pallas kernel here")


# speedup vs baseline: 8.5491x; 8.5491x over previous
import functools

import jax
import jax.numpy as jnp
from jax import lax
from jax.experimental import pallas as pl
from jax.experimental.pallas import tpu as pltpu
from jax.experimental.pallas import tpu_sc as plsc

NC = 2
NS = 16
NW = NC * NS
LANES = 16

_N = 10000
_E = 320000
C = 80
EPT = _E // NW
STEPS = EPT // C
N_PAD = 10112
RPS = N_PAD // NS
DEG_W = 16


@functools.lru_cache(maxsize=None)
def _build_sc_agg(feat_w: int, with_deg: bool):
  mesh = plsc.VectorSubcoreMesh(core_axis_name="c", subcore_axis_name="s")
  out_type = [jax.ShapeDtypeStruct((NC, N_PAD, feat_w), jnp.float32)]
  scratch = [
      pltpu.VMEM((STEPS, C), jnp.int32),
      pltpu.VMEM((STEPS, C), jnp.int32),
      pltpu.VMEM((C, feat_w), jnp.float32),
      pltpu.VMEM_SHARED((N_PAD, feat_w), jnp.float32),
  ]
  if with_deg:
    out_type.append(jax.ShapeDtypeStruct((NC, N_PAD, DEG_W), jnp.float32))
    scratch += [
        pltpu.VMEM((C, DEG_W), jnp.float32),
        pltpu.VMEM_SHARED((N_PAD, DEG_W), jnp.float32),
    ]

  def body(*refs):
    if with_deg:
      (feat_h, src_h, dst_h, out_h, deg_h, srcv, dstv, rows, acc, ones,
       dacc) = refs
    else:
      feat_h, src_h, dst_h, out_h, srcv, dstv, rows, acc = refs
      deg_h = ones = dacc = None

    cid = lax.axis_index("c")
    sid = lax.axis_index("s")
    wid = sid * NC + cid
    base_r = sid * RPS

    zeros = jnp.zeros((LANES,), jnp.float32)

    @pl.loop(0, C)
    def _(i):
      @pl.loop(0, feat_w // LANES)
      def _(j):
        rows[i, pl.ds(j * LANES, LANES)] = zeros

    for t in range(RPS // C):
      pltpu.sync_copy(rows, acc.at[pl.ds(base_r + t * C, C)])
    rem = RPS % C
    if rem:
      pltpu.sync_copy(rows.at[pl.ds(0, rem)],
                      acc.at[pl.ds(base_r + (RPS // C) * C, rem)])

    if with_deg:
      @pl.loop(0, C)
      def _(i):
        ones[i, pl.ds(0, LANES)] = zeros

      for t in range(RPS // C):
        pltpu.sync_copy(ones, dacc.at[pl.ds(base_r + t * C, C)])
      if rem:
        pltpu.sync_copy(ones.at[pl.ds(0, rem)],
                        dacc.at[pl.ds(base_r + (RPS // C) * C, rem)])

      one = jnp.ones((LANES,), jnp.float32)

      @pl.loop(0, C)
      def _(i):
        ones[i, pl.ds(0, LANES)] = one

    pltpu.sync_copy(src_h.at[wid], srcv)
    pltpu.sync_copy(dst_h.at[wid], dstv)

    plsc.subcore_barrier()

    @pl.loop(0, STEPS)
    def _(g):
      pltpu.sync_copy(feat_h.at[srcv.at[g]], rows)
      pltpu.sync_copy(rows, acc.at[dstv.at[g]], add=True)
      if with_deg:
        pltpu.sync_copy(ones, dacc.at[dstv.at[g]], add=True)

    plsc.subcore_barrier()

    pltpu.sync_copy(acc.at[pl.ds(base_r, RPS)],
                    out_h.at[cid, pl.ds(base_r, RPS)])
    if with_deg:
      pltpu.sync_copy(dacc.at[pl.ds(base_r, RPS)],
                      deg_h.at[cid, pl.ds(base_r, RPS)])

  return pl.kernel(
      body, out_type=out_type, mesh=mesh, scratch_types=scratch,
      compiler_params=pltpu.CompilerParams(use_tc_tiling_on_sc=False))


def _tc_layer1(x, pf, pd, w_self1, w_neigh1, b1, w_self2, w_neigh2):
  n, d_in = x.shape
  d_hid = w_self1.shape[1]
  d_out = w_self2.shape[1]
  blk = 2000

  def body(x_r, pf_r, pd_r, ws1_r, wn1_r, b1_r, ws2_r, wn2_r, p2_r, s2_r):
    agg = pf_r[0] + pf_r[1]
    deg = pd_r[0, :, 0:1] + pd_r[1, :, 0:1]
    hn = agg / jnp.maximum(deg, 1.0)
    h = (jnp.dot(x_r[...], ws1_r[...], preferred_element_type=jnp.float32)
         + jnp.dot(hn, wn1_r[...], preferred_element_type=jnp.float32)
         + b1_r[...])
    h = jnp.maximum(h, 0.0)
    p2_r[...] = jnp.dot(h, wn2_r[...], preferred_element_type=jnp.float32)
    s2_r[...] = jnp.dot(h, ws2_r[...], preferred_element_type=jnp.float32)

  return pl.pallas_call(
      body,
      grid=(n // blk,),
      in_specs=[
          pl.BlockSpec((blk, d_in), lambda i: (i, 0)),
          pl.BlockSpec((NC, blk, d_in), lambda i: (0, i, 0)),
          pl.BlockSpec((NC, blk, DEG_W), lambda i: (0, i, 0)),
          pl.BlockSpec((d_in, d_hid), lambda i: (0, 0)),
          pl.BlockSpec((d_in, d_hid), lambda i: (0, 0)),
          pl.BlockSpec((1, d_hid), lambda i: (0, 0)),
          pl.BlockSpec((d_hid, d_out), lambda i: (0, 0)),
          pl.BlockSpec((d_hid, d_out), lambda i: (0, 0)),
      ],
      out_specs=[
          pl.BlockSpec((blk, d_out), lambda i: (i, 0)),
          pl.BlockSpec((blk, d_out), lambda i: (i, 0)),
      ],
      out_shape=[
          jax.ShapeDtypeStruct((n, d_out), jnp.float32),
          jax.ShapeDtypeStruct((n, d_out), jnp.float32),
      ],
  )(x, pf, pd, w_self1, w_neigh1, b1.reshape(1, -1), w_self2, w_neigh2)


def _tc_layer2(s2, pf2, pd, b2):
  n, d_out = s2.shape
  blk = 2000

  def body(s2_r, pf2_r, pd_r, b2_r, o_r):
    agg = pf2_r[0] + pf2_r[1]
    deg = pd_r[0, :, 0:1] + pd_r[1, :, 0:1]
    o_r[...] = s2_r[...] + agg / jnp.maximum(deg, 1.0) + b2_r[...]

  return pl.pallas_call(
      body,
      grid=(n // blk,),
      in_specs=[
          pl.BlockSpec((blk, d_out), lambda i: (i, 0)),
          pl.BlockSpec((NC, blk, d_out), lambda i: (0, i, 0)),
          pl.BlockSpec((NC, blk, DEG_W), lambda i: (0, i, 0)),
          pl.BlockSpec((1, d_out), lambda i: (0, 0)),
      ],
      out_specs=pl.BlockSpec((blk, d_out), lambda i: (i, 0)),
      out_shape=jax.ShapeDtypeStruct((n, d_out), jnp.float32),
  )(s2, pf2, pd, b2.reshape(1, -1))


def kernel(x, edge_index, W_self1, W_neigh1, b1, W_self2, W_neigh2, b2):
  src = edge_index[0].reshape(NW, STEPS, C)
  dst = edge_index[1].reshape(NW, STEPS, C)

  pf1, pd = _build_sc_agg(x.shape[1], True)(x, src, dst)
  p2, s2 = _tc_layer1(x, pf1, pd, W_self1, W_neigh1, b1, W_self2, W_neigh2)
  (pf2,) = _build_sc_agg(p2.shape[1], False)(p2, src, dst)
  return _tc_layer2(s2, pf2, pd, b2)


# speedup vs baseline: 11.2461x; 1.3155x over previous
import dataclasses
import functools

import jax
import jax.numpy as jnp
from jax import lax
from jax.experimental import pallas as pl
from jax.experimental.pallas import tpu as pltpu
from jax.experimental.pallas import tpu_sc as plsc

NC = 2
NS = 16
NW = NC * NS
LANES = 16

_N = 10000
_E = 320000
C = 80
EPT = _E // NW
EPT_PAD = 10000
STEPS = EPT_PAD // C
N_PAD = 10000
RPS = N_PAD // NS
TBLK = 2000
NBLK = _N // TBLK


@functools.lru_cache(maxsize=None)
def _build_sc_agg(feat_w: int, with_deg: bool):
  mesh = plsc.VectorSubcoreMesh(core_axis_name="c", subcore_axis_name="s")
  out_type = [jax.ShapeDtypeStruct((NC, N_PAD, feat_w), jnp.float32)]
  scratch = [
      pltpu.VMEM((STEPS, C), jnp.int32),
      pltpu.VMEM((STEPS, C), jnp.int32),
      pltpu.VMEM((C, feat_w), jnp.float32),
      pltpu.VMEM((C, feat_w), jnp.float32),
      pltpu.VMEM_SHARED((N_PAD, feat_w), jnp.float32),
      pltpu.SemaphoreType.DMA,
      pltpu.SemaphoreType.DMA,
  ]
  if with_deg:
    scratch.append(pltpu.VMEM((N_PAD,), jnp.float32))
    out_type.append(jax.ShapeDtypeStruct((NBLK, NW, TBLK), jnp.float32))

  def body(*refs):
    if with_deg:
      (feat_h, src_h, dst_h, out_h, deg_h, srcv, dstv, rows0, rows1, acc,
       gsem0, gsem1, hist) = refs
    else:
      (feat_h, src_h, dst_h, out_h, srcv, dstv, rows0, rows1, acc,
       gsem0, gsem1) = refs
      deg_h = hist = None

    cid = lax.axis_index("c")
    sid = lax.axis_index("s")
    wid = sid * NC + cid
    base_r = sid * RPS

    zeros = jnp.zeros((LANES,), jnp.float32)
    onesv = jnp.ones((LANES,), jnp.float32)

    @pl.loop(0, C)
    def _(i):
      @pl.loop(0, feat_w // LANES)
      def _(j):
        rows0[i, pl.ds(j * LANES, LANES)] = zeros

    if with_deg:
      @pl.loop(0, N_PAD // LANES)
      def _(i):
        hist[pl.ds(i * LANES, LANES)] = zeros

    ncop = RPS // C
    rem = RPS % C
    for t in range(ncop):
      pltpu.sync_copy(rows0, acc.at[pl.ds(base_r + t * C, C)])
    if rem:
      pltpu.sync_copy(rows0.at[pl.ds(0, rem)],
                      acc.at[pl.ds(base_r + ncop * C, rem)])

    pltpu.sync_copy(src_h.at[wid], srcv)
    pltpu.sync_copy(dst_h.at[wid], dstv)

    plsc.subcore_barrier()

    def scat(rbuf, g):
      pltpu.sync_copy(rbuf, acc.at[dstv.at[g]], add=True)
      if with_deg:
        @pl.loop(0, C // LANES)
        def _(j):
          idx = dstv[g, pl.ds(j * LANES, LANES)]
          plsc.addupdate_scatter(hist, [idx], onesv)

    @pl.loop(0, STEPS // 2)
    def _(k):
      g = 2 * k
      d0 = pltpu.async_copy(feat_h.at[srcv.at[g]], rows0, gsem0)
      d1 = pltpu.async_copy(feat_h.at[srcv.at[g + 1]], rows1, gsem1)
      d0.wait()
      scat(rows0, g)
      d1.wait()
      scat(rows1, g + 1)

    if STEPS % 2:
      pltpu.sync_copy(feat_h.at[srcv.at[STEPS - 1]], rows0)
      scat(rows0, STEPS - 1)

    plsc.subcore_barrier()

    pltpu.sync_copy(acc.at[pl.ds(base_r, RPS)],
                    out_h.at[cid, pl.ds(base_r, RPS)])
    if with_deg:
      for b in range(NBLK):
        pltpu.sync_copy(hist.at[pl.ds(b * TBLK, TBLK)], deg_h.at[b, wid])

  cp = pltpu.CompilerParams(use_tc_tiling_on_sc=False)
  if with_deg and "needs_layout_passes" in pltpu.CompilerParams.__dataclass_fields__:
    cp = dataclasses.replace(cp, needs_layout_passes=False)
  return pl.kernel(
      body, out_type=out_type, mesh=mesh, scratch_types=scratch,
      compiler_params=cp)


def _tc_layer1(x, pf, pd, w_self1, w_neigh1, b1, w_self2, w_neigh2):
  n, d_in = x.shape
  d_hid = w_self1.shape[1]
  d_out = w_self2.shape[1]
  blk = 2000

  def body(x_r, pf_r, pd_r, ws1_r, wn1_r, b1_r, ws2_r, wn2_r, p2_r, s2_r):
    agg = pf_r[0] + pf_r[1]
    deg = jnp.sum(pd_r[0], axis=0)[:, None]
    hn = agg / jnp.maximum(deg, 1.0)
    h = (jnp.dot(x_r[...], ws1_r[...], preferred_element_type=jnp.float32)
         + jnp.dot(hn, wn1_r[...], preferred_element_type=jnp.float32)
         + b1_r[...])
    h = jnp.maximum(h, 0.0)
    p2_r[...] = jnp.dot(h, wn2_r[...], preferred_element_type=jnp.float32)
    s2_r[...] = jnp.dot(h, ws2_r[...], preferred_element_type=jnp.float32)

  return pl.pallas_call(
      body,
      grid=(n // blk,),
      in_specs=[
          pl.BlockSpec((blk, d_in), lambda i: (i, 0)),
          pl.BlockSpec((NC, blk, d_in), lambda i: (0, i, 0)),
          pl.BlockSpec((1, NW, blk), lambda i: (i, 0, 0)),
          pl.BlockSpec((d_in, d_hid), lambda i: (0, 0)),
          pl.BlockSpec((d_in, d_hid), lambda i: (0, 0)),
          pl.BlockSpec((1, d_hid), lambda i: (0, 0)),
          pl.BlockSpec((d_hid, d_out), lambda i: (0, 0)),
          pl.BlockSpec((d_hid, d_out), lambda i: (0, 0)),
      ],
      out_specs=[
          pl.BlockSpec((blk, d_out), lambda i: (i, 0)),
          pl.BlockSpec((blk, d_out), lambda i: (i, 0)),
      ],
      out_shape=[
          jax.ShapeDtypeStruct((n, d_out), jnp.float32),
          jax.ShapeDtypeStruct((n, d_out), jnp.float32),
      ],
  )(x, pf, pd, w_self1, w_neigh1, b1.reshape(1, -1), w_self2, w_neigh2)


def _tc_layer2(s2, pf2, pd, b2):
  n, d_out = s2.shape
  blk = 2000

  def body(s2_r, pf2_r, pd_r, b2_r, o_r):
    agg = pf2_r[0] + pf2_r[1]
    deg = jnp.sum(pd_r[0], axis=0)[:, None]
    o_r[...] = s2_r[...] + agg / jnp.maximum(deg, 1.0) + b2_r[...]

  return pl.pallas_call(
      body,
      grid=(n // blk,),
      in_specs=[
          pl.BlockSpec((blk, d_out), lambda i: (i, 0)),
          pl.BlockSpec((NC, blk, d_out), lambda i: (0, i, 0)),
          pl.BlockSpec((1, NW, blk), lambda i: (i, 0, 0)),
          pl.BlockSpec((1, d_out), lambda i: (0, 0)),
      ],
      out_specs=pl.BlockSpec((blk, d_out), lambda i: (i, 0)),
      out_shape=jax.ShapeDtypeStruct((n, d_out), jnp.float32),
  )(s2, pf2, pd, b2.reshape(1, -1))


def kernel(x, edge_index, W_self1, W_neigh1, b1, W_self2, W_neigh2, b2):
  npad = EPT_PAD - EPT
  if npad:
    src = jnp.concatenate(
        [edge_index[0].reshape(NW, EPT),
         jnp.zeros((NW, npad), jnp.int32)], axis=1).reshape(NW, STEPS, C)
    dst = jnp.concatenate(
        [edge_index[1].reshape(NW, EPT),
         jnp.broadcast_to(_N + jnp.arange(npad, dtype=jnp.int32),
                          (NW, npad))], axis=1).reshape(NW, STEPS, C)
  else:
    src = edge_index[0].reshape(NW, STEPS, C)
    dst = edge_index[1].reshape(NW, STEPS, C)

  pf1, pd = _build_sc_agg(x.shape[1], True)(x, src, dst)
  p2, s2 = _tc_layer1(x, pf1, pd, W_self1, W_neigh1, b1, W_self2, W_neigh2)
  (pf2,) = _build_sc_agg(p2.shape[1], False)(p2, src, dst)
  return _tc_layer2(s2, pf2, pd, b2)


# speedup vs baseline: 13.8969x; 1.2357x over previous
import dataclasses
import functools

import jax
import jax.numpy as jnp
from jax import lax
from jax.experimental import pallas as pl
from jax.experimental.pallas import tpu as pltpu
from jax.experimental.pallas import tpu_sc as plsc

NC = 2
NS = 16
NW = NC * NS
LANES = 16

_N = 10000
_E = 320000
C = 80
EPT = _E // NW
EPT_PAD = 10000
STEPS = EPT_PAD // C
N_PAD = 10000
RPS = N_PAD // NS
TBLK = 2000
NBLK = _N // TBLK


@functools.lru_cache(maxsize=None)
def _build_sc_agg(feat_w: int, with_deg: bool):
  mesh = plsc.VectorSubcoreMesh(core_axis_name="c", subcore_axis_name="s")
  out_type = [jax.ShapeDtypeStruct((NC, N_PAD, feat_w), jnp.float32)]
  scratch = [
      pltpu.VMEM((STEPS, C), jnp.int32),
      pltpu.VMEM((STEPS, C), jnp.int32),
      pltpu.VMEM((C, feat_w), jnp.float32),
      pltpu.VMEM((C, feat_w), jnp.float32),
      pltpu.VMEM_SHARED((N_PAD, feat_w), jnp.float32),
      pltpu.SemaphoreType.DMA,
      pltpu.SemaphoreType.DMA,
  ]
  if with_deg:
    scratch.append(pltpu.VMEM((N_PAD,), jnp.float32))
    out_type.append(jax.ShapeDtypeStruct((NBLK, NW, TBLK), jnp.float32))

  def body(*refs):
    if with_deg:
      (feat_h, src_h, dst_h, out_h, deg_h, srcv, dstv, rows0, rows1, acc,
       gsem0, gsem1, hist) = refs
    else:
      (feat_h, src_h, dst_h, out_h, srcv, dstv, rows0, rows1, acc,
       gsem0, gsem1) = refs
      deg_h = hist = None

    cid = lax.axis_index("c")
    sid = lax.axis_index("s")
    wid = sid * NC + cid
    base_r = sid * RPS

    zeros = jnp.zeros((LANES,), jnp.float32)
    onesv = jnp.ones((LANES,), jnp.float32)

    @pl.loop(0, C)
    def _(i):
      @pl.loop(0, feat_w // LANES)
      def _(j):
        rows0[i, pl.ds(j * LANES, LANES)] = zeros

    if with_deg:
      @pl.loop(0, N_PAD // LANES)
      def _(i):
        hist[pl.ds(i * LANES, LANES)] = zeros

    ncop = RPS // C
    rem = RPS % C
    for t in range(ncop):
      pltpu.sync_copy(rows0, acc.at[pl.ds(base_r + t * C, C)])
    if rem:
      pltpu.sync_copy(rows0.at[pl.ds(0, rem)],
                      acc.at[pl.ds(base_r + ncop * C, rem)])

    pltpu.sync_copy(src_h.at[wid], srcv)
    pltpu.sync_copy(dst_h.at[wid], dstv)

    plsc.subcore_barrier()

    def scat(rbuf, g):
      pltpu.sync_copy(rbuf, acc.at[dstv.at[g]], add=True)
      if with_deg:
        @pl.loop(0, C // LANES)
        def _(j):
          idx = dstv[g, pl.ds(j * LANES, LANES)]
          plsc.addupdate_scatter(hist, [idx], onesv)

    pltpu.async_copy(feat_h.at[srcv.at[0]], rows0, gsem0)

    @pl.loop(0, (STEPS - 1) // 2)
    def _(k):
      g = 2 * k
      pltpu.async_copy(feat_h.at[srcv.at[g + 1]], rows1, gsem1)
      pltpu.make_async_copy(feat_h.at[srcv.at[g]], rows0, gsem0).wait()
      scat(rows0, g)
      pltpu.async_copy(feat_h.at[srcv.at[g + 2]], rows0, gsem0)
      pltpu.make_async_copy(feat_h.at[srcv.at[g + 1]], rows1, gsem1).wait()
      scat(rows1, g + 1)

    pltpu.make_async_copy(feat_h.at[srcv.at[STEPS - 1]], rows0, gsem0).wait()
    scat(rows0, STEPS - 1)

    plsc.subcore_barrier()

    pltpu.sync_copy(acc.at[pl.ds(base_r, RPS)],
                    out_h.at[cid, pl.ds(base_r, RPS)])
    if with_deg:
      for b in range(NBLK):
        pltpu.sync_copy(hist.at[pl.ds(b * TBLK, TBLK)], deg_h.at[b, wid])

  cp = pltpu.CompilerParams(use_tc_tiling_on_sc=False)
  if with_deg and "needs_layout_passes" in pltpu.CompilerParams.__dataclass_fields__:
    cp = dataclasses.replace(cp, needs_layout_passes=False)
  return pl.kernel(
      body, out_type=out_type, mesh=mesh, scratch_types=scratch,
      compiler_params=cp)


def _tc_layer1(x, pf, pd, w_self1, w_neigh1, b1, w_self2, w_neigh2):
  n, d_in = x.shape
  d_hid = w_self1.shape[1]
  d_out = w_self2.shape[1]
  blk = 2000

  def body(x_r, pf_r, pd_r, ws1_r, wn1_r, b1_r, ws2_r, wn2_r, p2_r, s2_r):
    agg = pf_r[0] + pf_r[1]
    deg = jnp.sum(pd_r[0], axis=0)[:, None]
    hn = agg / jnp.maximum(deg, 1.0)
    h = (jnp.dot(x_r[...], ws1_r[...], preferred_element_type=jnp.float32)
         + jnp.dot(hn, wn1_r[...], preferred_element_type=jnp.float32)
         + b1_r[...])
    h = jnp.maximum(h, 0.0)
    p2_r[...] = jnp.dot(h, wn2_r[...], preferred_element_type=jnp.float32)
    s2_r[...] = jnp.dot(h, ws2_r[...], preferred_element_type=jnp.float32)

  return pl.pallas_call(
      body,
      grid=(n // blk,),
      in_specs=[
          pl.BlockSpec((blk, d_in), lambda i: (i, 0)),
          pl.BlockSpec((NC, blk, d_in), lambda i: (0, i, 0)),
          pl.BlockSpec((1, NW, blk), lambda i: (i, 0, 0)),
          pl.BlockSpec((d_in, d_hid), lambda i: (0, 0)),
          pl.BlockSpec((d_in, d_hid), lambda i: (0, 0)),
          pl.BlockSpec((1, d_hid), lambda i: (0, 0)),
          pl.BlockSpec((d_hid, d_out), lambda i: (0, 0)),
          pl.BlockSpec((d_hid, d_out), lambda i: (0, 0)),
      ],
      out_specs=[
          pl.BlockSpec((blk, d_out), lambda i: (i, 0)),
          pl.BlockSpec((blk, d_out), lambda i: (i, 0)),
      ],
      out_shape=[
          jax.ShapeDtypeStruct((n, d_out), jnp.float32),
          jax.ShapeDtypeStruct((n, d_out), jnp.float32),
      ],
  )(x, pf, pd, w_self1, w_neigh1, b1.reshape(1, -1), w_self2, w_neigh2)


def _tc_layer2(s2, pf2, pd, b2):
  n, d_out = s2.shape
  blk = 2000

  def body(s2_r, pf2_r, pd_r, b2_r, o_r):
    agg = pf2_r[0] + pf2_r[1]
    deg = jnp.sum(pd_r[0], axis=0)[:, None]
    o_r[...] = s2_r[...] + agg / jnp.maximum(deg, 1.0) + b2_r[...]

  return pl.pallas_call(
      body,
      grid=(n // blk,),
      in_specs=[
          pl.BlockSpec((blk, d_out), lambda i: (i, 0)),
          pl.BlockSpec((NC, blk, d_out), lambda i: (0, i, 0)),
          pl.BlockSpec((1, NW, blk), lambda i: (i, 0, 0)),
          pl.BlockSpec((1, d_out), lambda i: (0, 0)),
      ],
      out_specs=pl.BlockSpec((blk, d_out), lambda i: (i, 0)),
      out_shape=jax.ShapeDtypeStruct((n, d_out), jnp.float32),
  )(s2, pf2, pd, b2.reshape(1, -1))


def kernel(x, edge_index, W_self1, W_neigh1, b1, W_self2, W_neigh2, b2):
  npad = EPT_PAD - EPT
  if npad:
    src = jnp.concatenate(
        [edge_index[0].reshape(NW, EPT),
         jnp.zeros((NW, npad), jnp.int32)], axis=1).reshape(NW, STEPS, C)
    dst = jnp.concatenate(
        [edge_index[1].reshape(NW, EPT),
         jnp.broadcast_to(_N + jnp.arange(npad, dtype=jnp.int32),
                          (NW, npad))], axis=1).reshape(NW, STEPS, C)
  else:
    src = edge_index[0].reshape(NW, STEPS, C)
    dst = edge_index[1].reshape(NW, STEPS, C)

  pf1, pd = _build_sc_agg(x.shape[1], True)(x, src, dst)
  p2, s2 = _tc_layer1(x, pf1, pd, W_self1, W_neigh1, b1, W_self2, W_neigh2)
  (pf2,) = _build_sc_agg(p2.shape[1], False)(p2, src, dst)
  return _tc_layer2(s2, pf2, pd, b2)
